# baseline (speedup 1.0000x reference)
import functools

import jax
import jax.numpy as jnp
from jax import lax
from jax.experimental import pallas as pl
from jax.experimental.pallas import tpu as pltpu
from jax.experimental.pallas import tpu_sc as plsc

S = 30.0
M = 0.4

_B = 1024
_C = 100000

_NW = 32
_ROWS_PW = _B // _NW
_CHUNK = 10000
_CPR = _C // _CHUNK
_T = _ROWS_PW * _CPR
_NBUF = 4
_VREGS = _CHUNK // 16


def _splat_lane(vec16, lane):
    lane_vec = jnp.zeros((16,), jnp.int32) + lane
    return lax.gather(
        vec16,
        lane_vec[:, None],
        dimension_numbers=lax.GatherDimensionNumbers(
            offset_dims=(),
            collapsed_slice_dims=(0,),
            start_index_map=(0,),
        ),
        slice_sizes=(1,),
        mode=lax.GatherScatterMode.PROMISE_IN_BOUNDS,
    )


def _sc_body(cos_hbm, lab_hbm, out_hbm, labels_v, *bufs_and_sems):
    bufin = bufs_and_sems[:_NBUF]
    bufout = bufs_and_sems[_NBUF:2 * _NBUF]
    in_sems = bufs_and_sems[2 * _NBUF]
    out_sems = bufs_and_sems[2 * _NBUF + 1]

    wid = lax.axis_index("s") * 2 + lax.axis_index("c")
    base_row = wid * _ROWS_PW
    base_elem = base_row * _C

    pltpu.sync_copy(lab_hbm.at[pl.ds(base_row, _ROWS_PW)], labels_v)

    def start_in(t, b):
        pltpu.async_copy(
            cos_hbm.at[pl.ds(base_elem + t * _CHUNK, _CHUNK)],
            bufin[b],
            in_sems.at[b],
        )

    def wait_in(t, b):
        pltpu.make_async_copy(
            cos_hbm.at[pl.ds(base_elem + t * _CHUNK, _CHUNK)],
            bufin[b],
            in_sems.at[b],
        ).wait()

    def start_out(t, b):
        pltpu.async_copy(
            bufout[b],
            out_hbm.at[pl.ds(base_elem + t * _CHUNK, _CHUNK)],
            out_sems.at[b],
        )

    def wait_out(t, b):
        pltpu.make_async_copy(
            bufout[b],
            out_hbm.at[pl.ds(base_elem + t * _CHUNK, _CHUNK)],
            out_sems.at[b],
        ).wait()

    for b in range(_NBUF):
        start_in(b, b)

    iota16 = lax.iota(jnp.int32, 16)

    def round_body(g, _):
        for b in range(_NBUF):
            t = g * _NBUF + b
            row_local = t // _CPR
            col_base = (t % _CPR) * _CHUNK
            lab_group = labels_v[pl.ds((row_local // 16) * 16, 16)]
            lab_vec = _splat_lane(lab_group, row_local % 16)

            wait_in(t, b)

            @pl.when(g > 0)
            def _():
                wait_out(t - _NBUF, b)

            def vec_body(j, _):
                x = bufin[b][pl.ds(j * 16, 16)]
                cols = iota16 + (col_base + j * 16)
                y = x * S - jnp.where(cols == lab_vec, S * M, jnp.float32(0.0))
                bufout[b][pl.ds(j * 16, 16)] = y
                return 0

            lax.fori_loop(0, _VREGS, vec_body, 0)

            start_out(t, b)

            @pl.when(t + _NBUF < _T)
            def _():
                start_in(t + _NBUF, b)
        return 0

    lax.fori_loop(0, _T // _NBUF, round_body, 0)

    for b in range(_NBUF):
        wait_out(_T - _NBUF + b, b)


@jax.jit
def kernel(cosine, label):
    B, C = cosine.shape
    cos_flat = cosine.reshape(B * C)
    lab32 = label.astype(jnp.int32)

    mesh = plsc.VectorSubcoreMesh(core_axis_name="c", subcore_axis_name="s")
    out_flat = pl.kernel(
        _sc_body,
        mesh=mesh,
        out_type=jax.ShapeDtypeStruct((B * C,), jnp.float32),
        scratch_types=(
            [pltpu.VMEM((_ROWS_PW,), jnp.int32)]
            + [pltpu.VMEM((_CHUNK,), jnp.float32) for _ in range(2 * _NBUF)]
            + [pltpu.SemaphoreType.DMA((_NBUF,)), pltpu.SemaphoreType.DMA((_NBUF,))]
        ),
    )(cos_flat, lab32)
    return out_flat.reshape(B, C)
